# baseline (speedup 1.0000x reference)
import jax
import jax.numpy as jnp
from jax.experimental import pallas as pl
from jax.experimental.pallas import tpu as pltpu

_TM = 1024
_TR = 256


def _prep_kernel(table_ref, tbf_ref, lsec_ref):
    t = table_ref[...]
    tbf_ref[...] = t.astype(jnp.bfloat16)
    m = jnp.max(t, axis=-1, keepdims=True)
    lse = jnp.log(jnp.sum(jnp.exp(t - m), axis=-1, keepdims=True)) + m
    hi = lse.astype(jnp.bfloat16)
    hi32 = hi.astype(jnp.float32)
    lo32 = (lse - hi32).astype(jnp.bfloat16).astype(jnp.float32)
    col = jax.lax.broadcasted_iota(jnp.int32, lsec_ref.shape, 1)
    m0 = (col == 0).astype(jnp.float32)
    m1 = (col == 1).astype(jnp.float32)
    lsec_ref[...] = (hi32 * m0 + lo32 * m1).astype(jnp.bfloat16)


def _main_kernel(idx_ref, tgt_ref, tbf_ref, lsec_ref, logits_ref, nll_ref):
    idx = idx_ref[...]
    tgt = tgt_ref[...]
    tm = idx.shape[0]
    v = tbf_ref.shape[1]
    col = jax.lax.broadcasted_iota(jnp.int32, (tm, v), 1)
    oh = (col == idx).astype(jnp.bfloat16)
    acc = jnp.dot(oh, tbf_ref[...], preferred_element_type=jnp.float32)
    logits_ref[...] = acc
    acc2 = jnp.dot(oh, lsec_ref[...], preferred_element_type=jnp.float32)
    lse_tok = acc2[:, 0:1] + acc2[:, 1:2]
    tgt_logit = jnp.sum(jnp.where(col == tgt, acc, 0.0), axis=-1,
                        keepdims=True)
    nll_ref[...] = lse_tok - tgt_logit


def kernel(idx, table, targets):
    B, T = idx.shape
    V = table.shape[0]
    BT = B * T

    idx_flat = idx.reshape(BT, 1).astype(jnp.int32)
    tgt_flat = targets.reshape(BT, 1).astype(jnp.int32)
    table = table.astype(jnp.float32)

    n_prep = V // _TR
    tbf, lsec = pl.pallas_call(
        _prep_kernel,
        out_shape=(jax.ShapeDtypeStruct((V, V), jnp.bfloat16),
                   jax.ShapeDtypeStruct((V, 128), jnp.bfloat16)),
        grid=(n_prep,),
        in_specs=[pl.BlockSpec((_TR, V), lambda i: (i, 0))],
        out_specs=(pl.BlockSpec((_TR, V), lambda i: (i, 0)),
                   pl.BlockSpec((_TR, 128), lambda i: (i, 0))),
        compiler_params=pltpu.CompilerParams(
            dimension_semantics=("parallel",),
            vmem_limit_bytes=int(64 << 20)),
        cost_estimate=pl.CostEstimate(
            flops=3 * V * V,
            transcendentals=V * V,
            bytes_accessed=V * V * 4 + V * V * 2 + V * 128 * 2),
    )(table)

    n_blocks = BT // _TM
    cost = pl.CostEstimate(
        flops=2 * BT * V * (V + 128),
        transcendentals=0,
        bytes_accessed=(2 * BT * 4 + V * V * 2 + V * 128 * 2
                        + BT * V * 4 + BT * 4))
    logits, nll = pl.pallas_call(
        _main_kernel,
        out_shape=(jax.ShapeDtypeStruct((BT, V), jnp.float32),
                   jax.ShapeDtypeStruct((BT, 1), jnp.float32)),
        grid=(n_blocks,),
        in_specs=[pl.BlockSpec((_TM, 1), lambda i: (i, 0)),
                  pl.BlockSpec((_TM, 1), lambda i: (i, 0)),
                  pl.BlockSpec((V, V), lambda i: (0, 0)),
                  pl.BlockSpec((V, 128), lambda i: (0, 0))],
        out_specs=(pl.BlockSpec((_TM, V), lambda i: (i, 0)),
                   pl.BlockSpec((_TM, 1), lambda i: (i, 0))),
        compiler_params=pltpu.CompilerParams(
            dimension_semantics=("arbitrary",),
            vmem_limit_bytes=int(100 << 20)),
        cost_estimate=cost,
    )(idx_flat, tgt_flat, tbf, lsec)

    loss = jnp.sum(nll[:, 0]) / BT
    return logits, loss


# speedup vs baseline: 1.5520x; 1.5520x over previous
import jax
import jax.numpy as jnp
from jax.experimental import pallas as pl
from jax.experimental.pallas import tpu as pltpu

_TM = 128
_TR = 256


def _lse_kernel(table_ref, lse_ref):
    t = table_ref[...]
    m = jnp.max(t, axis=-1, keepdims=True)
    lse_ref[...] = jnp.log(jnp.sum(jnp.exp(t - m), axis=-1, keepdims=True)) + m


def _gather_kernel(idx_sref, idx_ref, tgt_ref, table3_ref, lse8_ref,
                   logits_ref, nll_ref, rows_ref):
    tm = logits_ref.shape[0]
    v = logits_ref.shape[1]
    base = pl.program_id(0) * tm
    for mi in range(tm):
        rows_ref[mi] = table3_ref[idx_sref[base + mi]]
    logits_ref[...] = rows_ref[...].reshape(tm, v)

    col = jax.lax.broadcasted_iota(jnp.int32, (tm, v), 1)
    lg = logits_ref[...]
    tgt_logit = jnp.sum(jnp.where(col == tgt_ref[...], lg, 0.0), axis=-1,
                        keepdims=True)
    lse_row = lse8_ref[0:1, :]
    lse_tok = jnp.sum((col == idx_ref[...]).astype(jnp.float32) * lse_row,
                      axis=-1, keepdims=True)
    nll_ref[...] = lse_tok - tgt_logit


def kernel(idx, table, targets):
    B, T = idx.shape
    V = table.shape[0]
    BT = B * T

    idx_flat = idx.reshape(BT).astype(jnp.int32)
    idx_col = idx_flat.reshape(BT, 1)
    tgt_col = targets.reshape(BT, 1).astype(jnp.int32)
    table = table.astype(jnp.float32)

    lse = pl.pallas_call(
        _lse_kernel,
        out_shape=jax.ShapeDtypeStruct((V, 1), jnp.float32),
        grid=(V // _TR,),
        in_specs=[pl.BlockSpec((_TR, V), lambda i: (i, 0))],
        out_specs=pl.BlockSpec((_TR, 1), lambda i: (i, 0)),
        compiler_params=pltpu.CompilerParams(
            dimension_semantics=("arbitrary",),
            vmem_limit_bytes=int(64 << 20)),
        cost_estimate=pl.CostEstimate(
            flops=3 * V * V,
            transcendentals=V * V,
            bytes_accessed=V * V * 4 + V * 4),
    )(table)

    lse8 = jnp.broadcast_to(lse.reshape(1, V), (8, V))
    table3 = table.reshape(V, 1, V)

    grid_spec = pltpu.PrefetchScalarGridSpec(
        num_scalar_prefetch=1,
        grid=(BT // _TM,),
        in_specs=[pl.BlockSpec((_TM, 1), lambda i, sref: (i, 0)),
                  pl.BlockSpec((_TM, 1), lambda i, sref: (i, 0)),
                  pl.BlockSpec((V, 1, V), lambda i, sref: (0, 0, 0)),
                  pl.BlockSpec((8, V), lambda i, sref: (0, 0))],
        out_specs=(pl.BlockSpec((_TM, V), lambda i, sref: (i, 0)),
                   pl.BlockSpec((_TM, 1), lambda i, sref: (i, 0))),
        scratch_shapes=[pltpu.VMEM((_TM, 1, V), jnp.float32)])
    logits, nll = pl.pallas_call(
        _gather_kernel,
        out_shape=(jax.ShapeDtypeStruct((BT, V), jnp.float32),
                   jax.ShapeDtypeStruct((BT, 1), jnp.float32)),
        grid_spec=grid_spec,
        compiler_params=pltpu.CompilerParams(
            dimension_semantics=("arbitrary",),
            vmem_limit_bytes=int(48 << 20)),
        cost_estimate=pl.CostEstimate(
            flops=4 * BT * V,
            transcendentals=0,
            bytes_accessed=V * V * 4 + BT * V * 4 + BT * 12),
    )(idx_flat, idx_col, tgt_col, table3, lse8)

    loss = jnp.sum(nll[:, 0]) / BT
    return logits, loss


# speedup vs baseline: 1.6829x; 1.0843x over previous
import jax
import jax.numpy as jnp
from jax.experimental import pallas as pl
from jax.experimental.pallas import tpu as pltpu

_TM = 128
_TR = 256


def _lse_kernel(table_ref, lse_ref, tbl3_ref):
    t = table_ref[...]
    m = jnp.max(t, axis=-1, keepdims=True)
    lse_ref[...] = jnp.log(jnp.sum(jnp.exp(t - m), axis=-1, keepdims=True)) + m
    tbl3_ref[...] = t.reshape(t.shape[0], 1, t.shape[1])


def _gather_kernel(idx_sref, idx_ref, tgt_ref, table3_ref, lse8_ref,
                   logits_ref, nll_ref, rows_ref):
    tm = logits_ref.shape[0]
    v = logits_ref.shape[1]
    base = pl.program_id(0) * tm
    for mi in range(tm):
        rows_ref[mi] = table3_ref[idx_sref[base + mi]]
    logits_ref[...] = rows_ref[...].reshape(tm, v)

    col = jax.lax.broadcasted_iota(jnp.int32, (tm, v), 1)
    lg = logits_ref[...]
    tgt_logit = jnp.sum(jnp.where(col == tgt_ref[...], lg, 0.0), axis=-1,
                        keepdims=True)
    lse_row = lse8_ref[0:1, :]
    lse_tok = jnp.sum((col == idx_ref[...]).astype(jnp.float32) * lse_row,
                      axis=-1, keepdims=True)
    nll_ref[...] = lse_tok - tgt_logit


def kernel(idx, table, targets):
    B, T = idx.shape
    V = table.shape[0]
    BT = B * T

    idx_flat = idx.reshape(BT).astype(jnp.int32)
    idx_col = idx_flat.reshape(BT, 1)
    tgt_col = targets.reshape(BT, 1).astype(jnp.int32)
    table = table.astype(jnp.float32)

    lse, table3 = pl.pallas_call(
        _lse_kernel,
        out_shape=(jax.ShapeDtypeStruct((V, 1), jnp.float32),
                   jax.ShapeDtypeStruct((V, 1, V), jnp.float32)),
        grid=(V // _TR,),
        in_specs=[pl.BlockSpec((_TR, V), lambda i: (i, 0))],
        out_specs=(pl.BlockSpec((_TR, 1), lambda i: (i, 0)),
                   pl.BlockSpec((_TR, 1, V), lambda i: (i, 0, 0))),
        compiler_params=pltpu.CompilerParams(
            dimension_semantics=("arbitrary",),
            vmem_limit_bytes=int(64 << 20)),
        cost_estimate=pl.CostEstimate(
            flops=3 * V * V,
            transcendentals=V * V,
            bytes_accessed=2 * V * V * 4 + V * 4),
    )(table)

    lse8 = jnp.broadcast_to(lse.reshape(1, V), (8, V))

    grid_spec = pltpu.PrefetchScalarGridSpec(
        num_scalar_prefetch=1,
        grid=(BT // _TM,),
        in_specs=[pl.BlockSpec((_TM, 1), lambda i, sref: (i, 0)),
                  pl.BlockSpec((_TM, 1), lambda i, sref: (i, 0)),
                  pl.BlockSpec((V, 1, V), lambda i, sref: (0, 0, 0)),
                  pl.BlockSpec((8, V), lambda i, sref: (0, 0))],
        out_specs=(pl.BlockSpec((_TM, V), lambda i, sref: (i, 0)),
                   pl.BlockSpec((_TM, 1), lambda i, sref: (i, 0))),
        scratch_shapes=[pltpu.VMEM((_TM, 1, V), jnp.float32)])
    logits, nll = pl.pallas_call(
        _gather_kernel,
        out_shape=(jax.ShapeDtypeStruct((BT, V), jnp.float32),
                   jax.ShapeDtypeStruct((BT, 1), jnp.float32)),
        grid_spec=grid_spec,
        compiler_params=pltpu.CompilerParams(
            dimension_semantics=("arbitrary",),
            vmem_limit_bytes=int(48 << 20)),
        cost_estimate=pl.CostEstimate(
            flops=4 * BT * V,
            transcendentals=0,
            bytes_accessed=V * V * 4 + BT * V * 4 + BT * 12),
    )(idx_flat, idx_col, tgt_col, table3, lse8)

    loss = jnp.sum(nll[:, 0]) / BT
    return logits, loss


# speedup vs baseline: 2.0536x; 1.2202x over previous
import jax
import jax.numpy as jnp
from jax.experimental import pallas as pl
from jax.experimental.pallas import tpu as pltpu

_TM = 128
_TR = 256


def _make_kernel(n_conv, tm, tr, v):
    ve = v + 128

    def _kernel(idx_sref, tgt_ref, tab_ref, logits_ref, nll_ref,
                tbl3_ref, rows_ref, lsec_ref):
        b = pl.program_id(0)

        @pl.when(b < n_conv)
        def _convert():
            t = tab_ref[...]
            m = jnp.max(t, axis=-1, keepdims=True)
            lse = jnp.log(jnp.sum(jnp.exp(t - m), axis=-1, keepdims=True)) + m
            lse_c = lse * jnp.ones((1, 128), jnp.float32)
            t_ext = jnp.concatenate([t, lse_c], axis=1)
            tbl3_ref[pl.ds(b * tr, tr)] = t_ext.reshape(tr, 1, ve)

        @pl.when(b >= n_conv)
        def _gather():
            base = (b - n_conv) * tm
            for mi in range(tm):
                rows_ref[mi] = tbl3_ref[idx_sref[base + mi]]
            logits_ref[...] = rows_ref[:, :, :v].reshape(tm, v)
            lsec_ref[...] = rows_ref[:, :, v:].reshape(tm, 128)

            col = jax.lax.broadcasted_iota(jnp.int32, (tm, v), 1)
            lg = logits_ref[...]
            tgt_logit = jnp.sum(jnp.where(col == tgt_ref[...], lg, 0.0),
                                axis=-1, keepdims=True)
            nll_ref[...] = lsec_ref[:, 0:1] - tgt_logit

    return _kernel


def kernel(idx, table, targets):
    B, T = idx.shape
    V = table.shape[0]
    BT = B * T
    VE = V + 128
    n_conv = V // _TR
    n_tok = BT // _TM

    idx_flat = idx.reshape(BT).astype(jnp.int32)
    tgt_col = targets.reshape(BT, 1).astype(jnp.int32)
    table = table.astype(jnp.float32)

    grid_spec = pltpu.PrefetchScalarGridSpec(
        num_scalar_prefetch=1,
        grid=(n_conv + n_tok,),
        in_specs=[
            pl.BlockSpec((_TM, 1),
                         lambda i, sref: (jnp.maximum(i - n_conv, 0), 0)),
            pl.BlockSpec((_TR, V),
                         lambda i, sref: (jnp.minimum(i, n_conv - 1), 0)),
        ],
        out_specs=(
            pl.BlockSpec((_TM, V),
                         lambda i, sref: (jnp.maximum(i - n_conv, 0), 0)),
            pl.BlockSpec((_TM, 1),
                         lambda i, sref: (jnp.maximum(i - n_conv, 0), 0)),
        ),
        scratch_shapes=[pltpu.VMEM((V, 1, VE), jnp.float32),
                        pltpu.VMEM((_TM, 1, VE), jnp.float32),
                        pltpu.VMEM((_TM, 128), jnp.float32)])
    logits, nll = pl.pallas_call(
        _make_kernel(n_conv, _TM, _TR, V),
        out_shape=(jax.ShapeDtypeStruct((BT, V), jnp.float32),
                   jax.ShapeDtypeStruct((BT, 1), jnp.float32)),
        grid_spec=grid_spec,
        compiler_params=pltpu.CompilerParams(
            dimension_semantics=("arbitrary",),
            vmem_limit_bytes=int(56 << 20)),
        cost_estimate=pl.CostEstimate(
            flops=6 * BT * V,
            transcendentals=V * V,
            bytes_accessed=V * V * 4 + BT * V * 4 + BT * 12),
    )(idx_flat, tgt_col, table)

    loss = jnp.sum(nll[:, 0]) / BT
    return logits, loss


# speedup vs baseline: 2.6450x; 1.2880x over previous
import jax
import jax.numpy as jnp
from jax.experimental import pallas as pl
from jax.experimental.pallas import tpu as pltpu

_TM = 256
_TR = 256


def _make_kernel(n_conv, tm, tr, v):
    ve = v + 128

    def _kernel(idx_sref, tgt_ref, tab_ref, logits_ref, nll_ref,
                tbl3_ref, rows_ref, lsec_ref):
        b = pl.program_id(0)

        @pl.when(b < n_conv)
        def _convert():
            t = tab_ref[...]
            m = jnp.max(t, axis=-1, keepdims=True)
            lse = jnp.log(jnp.sum(jnp.exp(t - m), axis=-1, keepdims=True)) + m
            lse_c = lse * jnp.ones((1, 128), jnp.float32)
            t_ext = jnp.concatenate([t, lse_c], axis=1)
            tbl3_ref[pl.ds(b * tr, tr)] = t_ext.reshape(tr, 1, ve)

        @pl.when(b >= n_conv)
        def _gather():
            base = (b - n_conv) * tm
            for mi in range(tm):
                rows_ref[mi] = tbl3_ref[idx_sref[base + mi]]
            logits_ref[...] = rows_ref[:, :, :v].reshape(tm, v)
            lsec_ref[...] = rows_ref[:, :, v:].reshape(tm, 128)

            col = jax.lax.broadcasted_iota(jnp.int32, (tm, v), 1)
            lg = logits_ref[...]
            tgt_logit = jnp.sum(jnp.where(col == tgt_ref[...], lg, 0.0),
                                axis=-1, keepdims=True)
            nll_ref[...] = lsec_ref[:, 0:1] - tgt_logit

    return _kernel


def kernel(idx, table, targets):
    B, T = idx.shape
    V = table.shape[0]
    BT = B * T
    VE = V + 128
    n_conv = V // _TR
    n_tok = BT // _TM

    idx_flat = idx.reshape(BT).astype(jnp.int32)
    tgt_col = targets.reshape(BT, 1).astype(jnp.int32)
    table = table.astype(jnp.float32)

    grid_spec = pltpu.PrefetchScalarGridSpec(
        num_scalar_prefetch=1,
        grid=(n_conv + n_tok,),
        in_specs=[
            pl.BlockSpec((_TM, 1),
                         lambda i, sref: (jnp.maximum(i - n_conv, 0), 0)),
            pl.BlockSpec((_TR, V),
                         lambda i, sref: (jnp.minimum(i, n_conv - 1), 0)),
        ],
        out_specs=(
            pl.BlockSpec((_TM, V),
                         lambda i, sref: (jnp.maximum(i - n_conv, 0), 0)),
            pl.BlockSpec((_TM, 1),
                         lambda i, sref: (jnp.maximum(i - n_conv, 0), 0)),
        ),
        scratch_shapes=[pltpu.VMEM((V, 1, VE), jnp.float32),
                        pltpu.VMEM((_TM, 1, VE), jnp.float32),
                        pltpu.VMEM((_TM, 128), jnp.float32)])
    logits, nll = pl.pallas_call(
        _make_kernel(n_conv, _TM, _TR, V),
        out_shape=(jax.ShapeDtypeStruct((BT, V), jnp.float32),
                   jax.ShapeDtypeStruct((BT, 1), jnp.float32)),
        grid_spec=grid_spec,
        compiler_params=pltpu.CompilerParams(
            dimension_semantics=("arbitrary",),
            vmem_limit_bytes=int(56 << 20)),
        cost_estimate=pl.CostEstimate(
            flops=6 * BT * V,
            transcendentals=V * V,
            bytes_accessed=V * V * 4 + BT * V * 4 + BT * 12),
    )(idx_flat, tgt_col, table)

    loss = jnp.sum(nll[:, 0]) / BT
    return logits, loss


# speedup vs baseline: 3.0846x; 1.1662x over previous
import jax
import jax.numpy as jnp
from jax.experimental import pallas as pl
from jax.experimental.pallas import tpu as pltpu

_TM = 512
_TR = 256


def _make_kernel(n_conv, tm, tr, v):
    ve = v + 128

    def _kernel(idx_sref, tgt_ref, tab_ref, logits_ref, nll_ref,
                tbl3_ref, rows_ref, lsec_ref):
        b = pl.program_id(0)

        @pl.when(b < n_conv)
        def _convert():
            t = tab_ref[...]
            m = jnp.max(t, axis=-1, keepdims=True)
            lse = jnp.log(jnp.sum(jnp.exp(t - m), axis=-1, keepdims=True)) + m
            lse_c = lse * jnp.ones((1, 128), jnp.float32)
            t_ext = jnp.concatenate([t, lse_c], axis=1)
            tbl3_ref[pl.ds(b * tr, tr)] = t_ext.reshape(tr, 1, ve)

        @pl.when(b >= n_conv)
        def _gather():
            base = (b - n_conv) * tm
            for mi in range(tm):
                rows_ref[mi] = tbl3_ref[idx_sref[base + mi]]
            logits_ref[...] = rows_ref[:, :, :v].reshape(tm, v)
            lsec_ref[...] = rows_ref[:, :, v:].reshape(tm, 128)

            col = jax.lax.broadcasted_iota(jnp.int32, (tm, v), 1)
            lg = logits_ref[...]
            tgt_logit = jnp.sum(jnp.where(col == tgt_ref[...], lg, 0.0),
                                axis=-1, keepdims=True)
            nll_ref[...] = lsec_ref[:, 0:1] - tgt_logit

    return _kernel


def kernel(idx, table, targets):
    B, T = idx.shape
    V = table.shape[0]
    BT = B * T
    VE = V + 128
    n_conv = V // _TR
    n_tok = BT // _TM

    idx_flat = idx.reshape(BT).astype(jnp.int32)
    tgt_col = targets.reshape(BT, 1).astype(jnp.int32)
    table = table.astype(jnp.float32)

    grid_spec = pltpu.PrefetchScalarGridSpec(
        num_scalar_prefetch=1,
        grid=(n_conv + n_tok,),
        in_specs=[
            pl.BlockSpec((_TM, 1),
                         lambda i, sref: (jnp.maximum(i - n_conv, 0), 0)),
            pl.BlockSpec((_TR, V),
                         lambda i, sref: (jnp.minimum(i, n_conv - 1), 0)),
        ],
        out_specs=(
            pl.BlockSpec((_TM, V),
                         lambda i, sref: (jnp.maximum(i - n_conv, 0), 0)),
            pl.BlockSpec((_TM, 1),
                         lambda i, sref: (jnp.maximum(i - n_conv, 0), 0)),
        ),
        scratch_shapes=[pltpu.VMEM((V, 1, VE), jnp.float32),
                        pltpu.VMEM((_TM, 1, VE), jnp.float32),
                        pltpu.VMEM((_TM, 128), jnp.float32)])
    logits, nll = pl.pallas_call(
        _make_kernel(n_conv, _TM, _TR, V),
        out_shape=(jax.ShapeDtypeStruct((BT, V), jnp.float32),
                   jax.ShapeDtypeStruct((BT, 1), jnp.float32)),
        grid_spec=grid_spec,
        compiler_params=pltpu.CompilerParams(
            dimension_semantics=("arbitrary",),
            vmem_limit_bytes=int(62 << 20)),
        cost_estimate=pl.CostEstimate(
            flops=6 * BT * V,
            transcendentals=V * V,
            bytes_accessed=V * V * 4 + BT * V * 4 + BT * 12),
    )(idx_flat, tgt_col, table)

    loss = jnp.sum(nll[:, 0]) / BT
    return logits, loss
